# baseline (speedup 1.0000x reference)
import jax
import jax.numpy as jnp
from jax.experimental import pallas as pl
from jax.experimental.pallas import tpu as pltpu

B, NS, NG, D, H = 2, 1024, 4096, 128, 4
DH = D // H
FF = D * 4
BLK_G = 512
NB = NG // BLK_G
VW = 64


def _ln(x, scale, bias, eps):
    r = 1.0 / x.shape[-1]
    m = jnp.sum(x, axis=-1, keepdims=True) * r
    v = jnp.sum(x * x, axis=-1, keepdims=True) * r - m * m
    rs = jax.lax.rsqrt(v + eps)
    return (x * rs - m * rs) * scale + bias


def _block_kernel(sys_ref, gene_ref, mask_ref,
                  wq_ref, wk_ref, wv_ref, wo_ref,
                  w1_ref, b1_ref, w2_ref, b2_ref,
                  gns_ref, gnb_ref, sns_ref, snb_ref,
                  ins_ref, inb_ref, outs_ref, outb_ref,
                  out_ref, kbig_s, v_s):
    g = pl.program_id(1)
    f32 = jnp.float32
    bf16 = jnp.bfloat16

    @pl.when(g == 0)
    def _compute_kv():
        sk = _ln(sys_ref[0], sns_ref[...], snb_ref[...], 1e-5).astype(bf16)
        k = jnp.dot(sk, wk_ref[...].astype(bf16), preferred_element_type=f32)
        v = jnp.dot(sk, wv_ref[...].astype(bf16), preferred_element_type=f32)
        kbig_s[...] = jnp.zeros((H * NS, D), bf16)
        v_s[...] = jnp.zeros((NS, H * VW), bf16)
        col = jax.lax.broadcasted_iota(jnp.int32, (NS, VW - DH), 1)
        ones_pad = (col == 0).astype(bf16)
        for h in range(H):
            kbig_s[h * NS:(h + 1) * NS, h * DH:(h + 1) * DH] = (
                k[:, h * DH:(h + 1) * DH].astype(bf16))
            v_s[:, h * VW:h * VW + DH] = v[:, h * DH:(h + 1) * DH].astype(bf16)
            v_s[:, h * VW + DH:(h + 1) * VW] = ones_pad

    gene = gene_ref[0]
    scale = 1.4426950408889634 / (DH ** 0.5)
    rg = 1.0 / D
    m_g = jnp.sum(gene, axis=-1, keepdims=True) * rg
    v_g = jnp.sum(gene * gene, axis=-1, keepdims=True) * rg - m_g * m_g
    rs_g = jax.lax.rsqrt(v_g + 1e-5)
    wq = wq_ref[...]
    wqs = wq * (gns_ref[...] * scale)[:, None]
    mmq = jnp.dot(gene.astype(bf16), wqs.astype(bf16),
                  preferred_element_type=f32)
    uq = jnp.sum(wqs, axis=0, keepdims=True)
    cq = jnp.sum(wq * (gnb_ref[...] * scale)[:, None], axis=0, keepdims=True)
    qb = (mmq * rs_g - (m_g * rs_g) * uq + cq).astype(bf16)
    mrs_g = m_g * rs_g
    gq = (gene * rs_g - mrs_g) * gns_ref[...] + gnb_ref[...]

    mvalid = mask_ref[0] > 0
    zero_b = jnp.zeros((BLK_G, NS), bf16)
    s_all = jax.lax.dot_general(
        qb, kbig_s[...], (((1,), (1,)), ((), ())),
        preferred_element_type=f32).astype(bf16)

    ctx_heads = []
    for h in range(H):
        p = jnp.where(mvalid, jnp.exp2(s_all[:, h * NS:(h + 1) * NS]), zero_b)
        o = jax.lax.dot_general(
            p, v_s[:, h * VW:(h + 1) * VW], (((1,), (0,)), ((), ())),
            preferred_element_type=f32)
        denom = o[:, DH:DH + 1]
        rcp = jnp.where(denom > 0, 1.0 / denom, 0.0)
        ctx_heads.append(o[:, :DH] * rcp)
    ctx = jnp.concatenate(ctx_heads, axis=1)

    o = jnp.dot(ctx.astype(bf16), wo_ref[...].astype(bf16),
                preferred_element_type=f32)
    x = _ln(gq + o, ins_ref[...], inb_ref[...], 0.1)
    h1 = (jnp.dot(x.astype(bf16), w1_ref[...].astype(bf16),
                  preferred_element_type=f32) + b1_ref[...]).astype(bf16)
    g1 = jax.nn.gelu(h1)
    ff = jnp.dot(g1, w2_ref[...].astype(bf16),
                 preferred_element_type=f32) + b2_ref[...]
    y = _ln(x + ff, outs_ref[...], outb_ref[...], 0.1)
    out_ref[0] = gene + y


@jax.jit
def kernel(system_embedding, gene_embedding, sys2gene_mask, Wq, Wk, Wv, Wo,
           W1, b1, W2, b2, gene_norm_s, gene_norm_b, sys_norm_s, sys_norm_b,
           inner_s, inner_b, outer_s, outer_b):
    full = lambda shape: pl.BlockSpec(shape, lambda b, g: (0,) * len(shape))
    out = pl.pallas_call(
        _block_kernel,
        grid=(B, NB),
        in_specs=[
            pl.BlockSpec((1, NS, D), lambda b, g: (b, 0, 0)),
            pl.BlockSpec((1, BLK_G, D), lambda b, g: (b, g, 0)),
            pl.BlockSpec((1, BLK_G, NS), lambda b, g: (b, g, 0)),
            full((D, D)), full((D, D)), full((D, D)), full((D, D)),
            full((D, FF)), full((FF,)), full((FF, D)), full((D,)),
            full((D,)), full((D,)), full((D,)), full((D,)),
            full((D,)), full((D,)), full((D,)), full((D,)),
        ],
        out_specs=pl.BlockSpec((1, BLK_G, D), lambda b, g: (b, g, 0)),
        out_shape=jax.ShapeDtypeStruct((B, NG, D), jnp.float32),
        scratch_shapes=[
            pltpu.VMEM((H * NS, D), jnp.bfloat16),
            pltpu.VMEM((NS, H * VW), jnp.bfloat16),
        ],
        compiler_params=pltpu.CompilerParams(
            dimension_semantics=("parallel", "arbitrary")),
    )(system_embedding, gene_embedding, sys2gene_mask,
      Wq, Wk, Wv, Wo, W1, b1, W2, b2,
      gene_norm_s, gene_norm_b, sys_norm_s, sys_norm_b,
      inner_s, inner_b, outer_s, outer_b)
    return out


# speedup vs baseline: 1.0772x; 1.0772x over previous
import jax
import jax.numpy as jnp
from jax.experimental import pallas as pl
from jax.experimental.pallas import tpu as pltpu

B, NS, NG, D, H = 2, 1024, 4096, 128, 4
DH = D // H
FF = D * 4
BLK_G = 1024
NB = NG // BLK_G
VW = 64


def _ln(x, scale, bias, eps):
    r = 1.0 / x.shape[-1]
    m = jnp.sum(x, axis=-1, keepdims=True) * r
    v = jnp.sum(x * x, axis=-1, keepdims=True) * r - m * m
    rs = jax.lax.rsqrt(v + eps)
    return (x * rs - m * rs) * scale + bias


def _block_kernel(sys_ref, gene_ref, mask_ref,
                  wq_ref, wk_ref, wv_ref, wo_ref,
                  w1_ref, b1_ref, w2_ref, b2_ref,
                  gns_ref, gnb_ref, sns_ref, snb_ref,
                  ins_ref, inb_ref, outs_ref, outb_ref,
                  out_ref, kbig_s, v_s):
    g = pl.program_id(1)
    f32 = jnp.float32
    bf16 = jnp.bfloat16

    @pl.when(g == 0)
    def _compute_kv():
        sk = _ln(sys_ref[0], sns_ref[...], snb_ref[...], 1e-5).astype(bf16)
        k = jnp.dot(sk, wk_ref[...].astype(bf16), preferred_element_type=f32)
        v = jnp.dot(sk, wv_ref[...].astype(bf16), preferred_element_type=f32)
        kbig_s[...] = jnp.zeros((H * NS, D), bf16)
        v_s[...] = jnp.zeros((NS, H * VW), bf16)
        col = jax.lax.broadcasted_iota(jnp.int32, (NS, VW - DH), 1)
        ones_pad = (col == 0).astype(bf16)
        for h in range(H):
            kbig_s[h * NS:(h + 1) * NS, h * DH:(h + 1) * DH] = (
                k[:, h * DH:(h + 1) * DH].astype(bf16))
            v_s[:, h * VW:h * VW + DH] = v[:, h * DH:(h + 1) * DH].astype(bf16)
            v_s[:, h * VW + DH:(h + 1) * VW] = ones_pad

    gene = gene_ref[0]
    scale = 1.4426950408889634 / (DH ** 0.5)
    rg = 1.0 / D
    m_g = jnp.sum(gene, axis=-1, keepdims=True) * rg
    v_g = jnp.sum(gene * gene, axis=-1, keepdims=True) * rg - m_g * m_g
    rs_g = jax.lax.rsqrt(v_g + 1e-5)
    wq = wq_ref[...]
    wqs = wq * (gns_ref[...] * scale)[:, None]
    mmq = jnp.dot(gene.astype(bf16), wqs.astype(bf16),
                  preferred_element_type=f32)
    uq = jnp.sum(wqs, axis=0, keepdims=True)
    cq = jnp.sum(wq * (gnb_ref[...] * scale)[:, None], axis=0, keepdims=True)
    qb = (mmq * rs_g - (m_g * rs_g) * uq + cq).astype(bf16)
    mrs_g = m_g * rs_g
    gq = (gene * rs_g - mrs_g) * gns_ref[...] + gnb_ref[...]

    maskb = mask_ref[0].astype(bf16)
    s_all = jax.lax.dot_general(
        qb, kbig_s[...], (((1,), (1,)), ((), ())),
        preferred_element_type=f32).astype(bf16)

    ctx_heads = []
    for h in range(H):
        p = jnp.exp2(s_all[:, h * NS:(h + 1) * NS]) * maskb
        o = jax.lax.dot_general(
            p, v_s[:, h * VW:(h + 1) * VW], (((1,), (0,)), ((), ())),
            preferred_element_type=f32)
        denom = o[:, DH:DH + 1]
        rcp = jnp.where(denom > 0, 1.0 / denom, 0.0)
        ctx_heads.append(o[:, :DH] * rcp)
    ctx = jnp.concatenate(ctx_heads, axis=1)

    o = jnp.dot(ctx.astype(bf16), wo_ref[...].astype(bf16),
                preferred_element_type=f32)
    x = _ln(gq + o, ins_ref[...], inb_ref[...], 0.1)
    h1 = (jnp.dot(x.astype(bf16), w1_ref[...].astype(bf16),
                  preferred_element_type=f32) + b1_ref[...]).astype(bf16)
    g1 = jax.nn.gelu(h1)
    ff = jnp.dot(g1, w2_ref[...].astype(bf16),
                 preferred_element_type=f32) + b2_ref[...]
    y = _ln(x + ff, outs_ref[...], outb_ref[...], 0.1)
    out_ref[0] = gene + y


@jax.jit
def kernel(system_embedding, gene_embedding, sys2gene_mask, Wq, Wk, Wv, Wo,
           W1, b1, W2, b2, gene_norm_s, gene_norm_b, sys_norm_s, sys_norm_b,
           inner_s, inner_b, outer_s, outer_b):
    full = lambda shape: pl.BlockSpec(shape, lambda b, g: (0,) * len(shape))
    out = pl.pallas_call(
        _block_kernel,
        grid=(B, NB),
        in_specs=[
            pl.BlockSpec((1, NS, D), lambda b, g: (b, 0, 0)),
            pl.BlockSpec((1, BLK_G, D), lambda b, g: (b, g, 0)),
            pl.BlockSpec((1, BLK_G, NS), lambda b, g: (b, g, 0)),
            full((D, D)), full((D, D)), full((D, D)), full((D, D)),
            full((D, FF)), full((FF,)), full((FF, D)), full((D,)),
            full((D,)), full((D,)), full((D,)), full((D,)),
            full((D,)), full((D,)), full((D,)), full((D,)),
        ],
        out_specs=pl.BlockSpec((1, BLK_G, D), lambda b, g: (b, g, 0)),
        out_shape=jax.ShapeDtypeStruct((B, NG, D), jnp.float32),
        scratch_shapes=[
            pltpu.VMEM((H * NS, D), jnp.bfloat16),
            pltpu.VMEM((NS, H * VW), jnp.bfloat16),
        ],
        compiler_params=pltpu.CompilerParams(
            dimension_semantics=("parallel", "arbitrary")),
    )(system_embedding, gene_embedding, sys2gene_mask,
      Wq, Wk, Wv, Wo, W1, b1, W2, b2,
      gene_norm_s, gene_norm_b, sys_norm_s, sys_norm_b,
      inner_s, inner_b, outer_s, outer_b)
    return out
